# Initial kernel scaffold; baseline (speedup 1.0000x reference)
import functools

import jax
import jax.numpy as jnp
from jax.experimental import pallas as pl

_NEG = jnp.float32(-1e30)


def _body(mb, na, nd, nf, nbf, hid, ncls,
          atoms_ref, bonds_ref, edges_ref,
          w1a_ref, w1b_ref, b1_ref,
          w2a_ref, w2b_ref, b2_ref,
          woa_ref, wob_ref, bo_ref,
          wf_ref, bf_ref, out_ref):
    ndeg = nd + 1
    atoms = atoms_ref[...]
    edges = edges_ref[...]
    bonds = bonds_ref[...]

    valid = edges != -1
    degf = jnp.sum(valid.astype(jnp.float32), axis=2)
    deg_col = degf.reshape(mb * na, 1)
    nz_col = (deg_col > 0.0).astype(jnp.float32)

    bsum = bonds[:, :, 0:nbf]
    for d in range(1, nd):
        bsum = bsum + bonds[:, :, d * nbf:(d + 1) * nbf]
    b6 = bsum.reshape(mb * na, nbf)

    col = jax.lax.broadcasted_iota(jnp.int32, (mb, na, na), 2)
    onehots = []
    for d in range(nd):
        e_d = edges[:, :, d]
        hit = (e_d[:, :, None] == col) & valid[:, :, d][:, :, None]
        onehots.append(hit.astype(jnp.float32))
    adj = onehots[0]
    for d in range(1, nd):
        adj = adj + onehots[d]

    def bmm(a, b):
        return jax.lax.dot_general(
            a, b, (((2,), (1,)), ((0,), (0,))),
            preferred_element_type=jnp.float32)

    def conv(x, wa_ref, wb_ref, b_ref, in_dim):
        summed = x + bmm(adj, x)
        s = summed.reshape(mb * na, in_dim)
        y = (jnp.dot(s, wa_ref[...], preferred_element_type=jnp.float32)
             + jnp.dot(b6, wb_ref[...], preferred_element_type=jnp.float32)
             + b_ref[...])
        y = jnp.maximum(y, 0.0)
        out = jnp.zeros((mb * na, hid), jnp.float32)
        for d in range(ndeg):
            m = (deg_col == float(d)).astype(jnp.float32)
            out = out + m * y[:, d * hid:(d + 1) * hid]
        return out.reshape(mb, na, hid)

    def pool(x):
        p = x
        for d in range(nd):
            g = bmm(onehots[d], x)
            vmask = valid[:, :, d][:, :, None]
            p = jnp.maximum(p, jnp.where(vmask, g, _NEG))
        return p * nz_col.reshape(mb, na, 1)

    x1 = conv(atoms, w1a_ref, w1b_ref, b1_ref, nf)
    p1 = pool(x1)
    x2 = conv(p1, w2a_ref, w2b_ref, b2_ref, hid)
    p2 = pool(x2)

    f = jnp.tanh(jnp.dot(p2.reshape(mb * na, hid), woa_ref[...],
                         preferred_element_type=jnp.float32)
                 + jnp.dot(b6, wob_ref[...],
                           preferred_element_type=jnp.float32)
                 + bo_ref[...])
    f = f * nz_col
    fp = f.reshape(mb, na, hid).sum(axis=1)
    logits = (jnp.dot(fp, wf_ref[...], preferred_element_type=jnp.float32)
              + bf_ref[...])
    out_ref[...] = jax.nn.sigmoid(logits)


@jax.jit
def kernel(atoms, bonds, edges, W1, b1, W2, b2, Wo, bo, Wf, bf):
    bm, na, nf = atoms.shape
    nd = edges.shape[-1]
    nbf = bonds.shape[-1]
    ndeg, in1, hid = W1.shape
    ncls = Wf.shape[-1]
    mb = 8
    grid = (bm // mb,)

    bonds_r = bonds.reshape(bm, na, nd * nbf)
    w1c = W1.transpose(1, 0, 2).reshape(in1, ndeg * hid)
    w1a, w1b = w1c[:nf], w1c[nf:]
    b1c = b1.reshape(1, ndeg * hid)
    in2 = W2.shape[1]
    w2c = W2.transpose(1, 0, 2).reshape(in2, ndeg * hid)
    w2a, w2b = w2c[:hid], w2c[hid:]
    b2c = b2.reshape(1, ndeg * hid)
    woa, wob = Wo[:hid], Wo[hid:]
    bo2 = bo.reshape(1, hid)
    bf2 = bf.reshape(1, ncls)

    const = lambda *shape: pl.BlockSpec(shape, lambda i: (0,) * len(shape))
    return pl.pallas_call(
        functools.partial(_body, mb, na, nd, nf, nbf, hid, ncls),
        grid=grid,
        in_specs=[
            pl.BlockSpec((mb, na, nf), lambda i: (i, 0, 0)),
            pl.BlockSpec((mb, na, nd * nbf), lambda i: (i, 0, 0)),
            pl.BlockSpec((mb, na, nd), lambda i: (i, 0, 0)),
            const(nf, ndeg * hid),
            const(in1 - nf, ndeg * hid),
            const(1, ndeg * hid),
            const(hid, ndeg * hid),
            const(in2 - hid, ndeg * hid),
            const(1, ndeg * hid),
            const(hid, hid),
            const(in2 - hid, hid),
            const(1, hid),
            const(hid, ncls),
            const(1, ncls),
        ],
        out_specs=pl.BlockSpec((mb, ncls), lambda i: (i, 0)),
        out_shape=jax.ShapeDtypeStruct((bm, ncls), jnp.float32),
    )(atoms, bonds_r, edges, w1a, w1b, b1c, w2a, w2b, b2c,
      woa, wob, bo2, Wf, bf2)


# speedup vs baseline: 42.0584x; 42.0584x over previous
import functools

import jax
import jax.numpy as jnp
from jax.experimental import pallas as pl

def _body(mb, na, nd, nf, nbf, hid, ncls,
          atoms_ref, bonds_ref, edges_ref,
          w1a_ref, w1b_ref, b1_ref,
          w2a_ref, w2b_ref, b2_ref,
          woa_ref, wob_ref, bo_ref,
          wf_ref, bf_ref, out_ref):
    ndeg = nd + 1
    atoms = atoms_ref[...]
    edges = edges_ref[...]
    bonds = bonds_ref[...]

    valid = edges != -1
    degf = jnp.sum(valid.astype(jnp.float32), axis=2)
    deg_col = degf.reshape(mb * na, 1)
    nz_col = (deg_col > 0.0).astype(jnp.float32)

    bsum = bonds[:, :, 0:nbf]
    for d in range(1, nd):
        bsum = bsum + bonds[:, :, d * nbf:(d + 1) * nbf]
    b6 = bsum.reshape(mb * na, nbf)

    col = jax.lax.broadcasted_iota(jnp.int32, (mb, na, na), 2)
    onehots = []
    for d in range(nd):
        e_d = edges[:, :, d]
        hit = (e_d[:, :, None] == col) & valid[:, :, d][:, :, None]
        onehots.append(hit.astype(jnp.float32))
    adj = onehots[0]
    for d in range(1, nd):
        adj = adj + onehots[d]

    def bmm(a, b):
        return jax.lax.dot_general(
            a, b, (((2,), (1,)), ((0,), (0,))),
            preferred_element_type=jnp.float32, precision=jax.lax.Precision.HIGHEST)

    def conv(x, wa_ref, wb_ref, b_ref, in_dim):
        summed = x + bmm(adj, x)
        s = summed.reshape(mb * na, in_dim)
        y = (jnp.dot(s, wa_ref[...], preferred_element_type=jnp.float32)
             + jnp.dot(b6, wb_ref[...], preferred_element_type=jnp.float32)
             + b_ref[...])
        y = jnp.maximum(y, 0.0)
        out = jnp.zeros((mb * na, hid), jnp.float32)
        for d in range(ndeg):
            m = (deg_col == float(d)).astype(jnp.float32)
            out = out + m * y[:, d * hid:(d + 1) * hid]
        return out.reshape(mb, na, hid)

    def pool(x):
        p = x
        for d in range(nd):
            g = bmm(onehots[d], x)
            vmask = valid[:, :, d][:, :, None]
            p = jnp.maximum(p, jnp.where(vmask, g, -1e30))
        return p * nz_col.reshape(mb, na, 1)

    x1 = conv(atoms, w1a_ref, w1b_ref, b1_ref, nf)
    p1 = pool(x1)
    x2 = conv(p1, w2a_ref, w2b_ref, b2_ref, hid)
    p2 = pool(x2)

    f = jnp.tanh(jnp.dot(p2.reshape(mb * na, hid), woa_ref[...],
                         preferred_element_type=jnp.float32)
                 + jnp.dot(b6, wob_ref[...],
                           preferred_element_type=jnp.float32)
                 + bo_ref[...])
    f = f * nz_col
    fp = f.reshape(mb, na, hid).sum(axis=1)
    logits = (jnp.dot(fp, wf_ref[...], preferred_element_type=jnp.float32)
              + bf_ref[...])
    out_ref[...] = jax.nn.sigmoid(logits)


@jax.jit
def kernel(atoms, bonds, edges, W1, b1, W2, b2, Wo, bo, Wf, bf):
    bm, na, nf = atoms.shape
    nd = edges.shape[-1]
    nbf = bonds.shape[-1]
    ndeg, in1, hid = W1.shape
    ncls = Wf.shape[-1]
    mb = 8
    grid = (bm // mb,)

    bonds_r = bonds.reshape(bm, na, nd * nbf)
    w1c = W1.transpose(1, 0, 2).reshape(in1, ndeg * hid)
    w1a, w1b = w1c[:nf], w1c[nf:]
    b1c = b1.reshape(1, ndeg * hid)
    in2 = W2.shape[1]
    w2c = W2.transpose(1, 0, 2).reshape(in2, ndeg * hid)
    w2a, w2b = w2c[:hid], w2c[hid:]
    b2c = b2.reshape(1, ndeg * hid)
    woa, wob = Wo[:hid], Wo[hid:]
    bo2 = bo.reshape(1, hid)
    bf2 = bf.reshape(1, ncls)

    const = lambda *shape: pl.BlockSpec(shape, lambda i: (0,) * len(shape))
    return pl.pallas_call(
        functools.partial(_body, mb, na, nd, nf, nbf, hid, ncls),
        grid=grid,
        in_specs=[
            pl.BlockSpec((mb, na, nf), lambda i: (i, 0, 0)),
            pl.BlockSpec((mb, na, nd * nbf), lambda i: (i, 0, 0)),
            pl.BlockSpec((mb, na, nd), lambda i: (i, 0, 0)),
            const(nf, ndeg * hid),
            const(in1 - nf, ndeg * hid),
            const(1, ndeg * hid),
            const(hid, ndeg * hid),
            const(in2 - hid, ndeg * hid),
            const(1, ndeg * hid),
            const(hid, hid),
            const(in2 - hid, hid),
            const(1, hid),
            const(hid, ncls),
            const(1, ncls),
        ],
        out_specs=pl.BlockSpec((mb, ncls), lambda i: (i, 0)),
        out_shape=jax.ShapeDtypeStruct((bm, ncls), jnp.float32),
    )(atoms, bonds_r, edges, w1a, w1b, b1c, w2a, w2b, b2c,
      woa, wob, bo2, Wf, bf2)


# speedup vs baseline: 43.0927x; 1.0246x over previous
import functools

import jax
import jax.numpy as jnp
from jax.experimental import pallas as pl

def _body(mb, na, nd, nf, nbf, hid, ncls,
          atoms_ref, bonds_ref, edges_ref,
          w1a_ref, w1b_ref, b1_ref,
          w2a_ref, w2b_ref, b2_ref,
          woa_ref, wob_ref, bo_ref,
          wf_ref, bf_ref, out_ref):
    ndeg = nd + 1
    atoms = atoms_ref[...]
    edges = edges_ref[...]
    bonds = bonds_ref[...]

    valid = edges != -1
    deg = jnp.sum(valid.astype(jnp.int32), axis=2)
    deg_col = deg.reshape(mb * na, 1)
    nz_col = (deg_col > 0).astype(jnp.float32)
    dlane = jax.lax.broadcasted_iota(jnp.int32, (mb * na, ndeg * hid), 1) // hid
    mask_cat = (dlane == deg_col).astype(jnp.float32)

    bsum = bonds[:, :, 0:nbf]
    for d in range(1, nd):
        bsum = bsum + bonds[:, :, d * nbf:(d + 1) * nbf]
    b6 = bsum.reshape(mb * na, nbf)

    col = jax.lax.broadcasted_iota(jnp.int32, (mb, na, na), 2)
    onehots = []
    for d in range(nd):
        e_d = edges[:, :, d]
        hit = (e_d[:, :, None] == col) & valid[:, :, d][:, :, None]
        onehots.append(hit.astype(jnp.float32))
    adj = onehots[0]
    for d in range(1, nd):
        adj = adj + onehots[d]
    oh_cat = jnp.concatenate(onehots, axis=1)

    def bmm(a, b):
        return jax.lax.dot_general(
            a, b, (((2,), (1,)), ((0,), (0,))),
            preferred_element_type=jnp.float32, precision=jax.lax.Precision.HIGHEST)

    def conv(x, wa_ref, wb_ref, b_ref, in_dim):
        summed = x + bmm(adj, x)
        s = summed.reshape(mb * na, in_dim)
        y = (jnp.dot(s, wa_ref[...], preferred_element_type=jnp.float32)
             + jnp.dot(b6, wb_ref[...], preferred_element_type=jnp.float32)
             + b_ref[...]) * mask_cat
        out = y[:, 0:hid]
        for d in range(1, ndeg):
            out = out + y[:, d * hid:(d + 1) * hid]
        return jnp.maximum(out, 0.0).reshape(mb, na, hid)

    def pool(x):
        g_all = bmm(oh_cat, x)
        p = x
        for d in range(nd):
            vmask = valid[:, :, d][:, :, None]
            g = g_all[:, d * na:(d + 1) * na, :]
            p = jnp.maximum(p, jnp.where(vmask, g, -1e30))
        return p * nz_col.reshape(mb, na, 1)

    x1 = conv(atoms, w1a_ref, w1b_ref, b1_ref, nf)
    p1 = pool(x1)
    x2 = conv(p1, w2a_ref, w2b_ref, b2_ref, hid)
    p2 = pool(x2)

    f = jnp.tanh(jnp.dot(p2.reshape(mb * na, hid), woa_ref[...],
                         preferred_element_type=jnp.float32)
                 + jnp.dot(b6, wob_ref[...],
                           preferred_element_type=jnp.float32)
                 + bo_ref[...])
    f = f * nz_col
    fp = f.reshape(mb, na, hid).sum(axis=1)
    logits = (jnp.dot(fp, wf_ref[...], preferred_element_type=jnp.float32)
              + bf_ref[...])
    out_ref[...] = jax.nn.sigmoid(logits)


@jax.jit
def kernel(atoms, bonds, edges, W1, b1, W2, b2, Wo, bo, Wf, bf):
    bm, na, nf = atoms.shape
    nd = edges.shape[-1]
    nbf = bonds.shape[-1]
    ndeg, in1, hid = W1.shape
    ncls = Wf.shape[-1]
    mb = 8
    grid = (bm // mb,)

    bonds_r = bonds.reshape(bm, na, nd * nbf)
    w1c = W1.transpose(1, 0, 2).reshape(in1, ndeg * hid)
    w1a, w1b = w1c[:nf], w1c[nf:]
    b1c = b1.reshape(1, ndeg * hid)
    in2 = W2.shape[1]
    w2c = W2.transpose(1, 0, 2).reshape(in2, ndeg * hid)
    w2a, w2b = w2c[:hid], w2c[hid:]
    b2c = b2.reshape(1, ndeg * hid)
    woa, wob = Wo[:hid], Wo[hid:]
    bo2 = bo.reshape(1, hid)
    bf2 = bf.reshape(1, ncls)

    const = lambda *shape: pl.BlockSpec(shape, lambda i: (0,) * len(shape))
    return pl.pallas_call(
        functools.partial(_body, mb, na, nd, nf, nbf, hid, ncls),
        grid=grid,
        in_specs=[
            pl.BlockSpec((mb, na, nf), lambda i: (i, 0, 0)),
            pl.BlockSpec((mb, na, nd * nbf), lambda i: (i, 0, 0)),
            pl.BlockSpec((mb, na, nd), lambda i: (i, 0, 0)),
            const(nf, ndeg * hid),
            const(in1 - nf, ndeg * hid),
            const(1, ndeg * hid),
            const(hid, ndeg * hid),
            const(in2 - hid, ndeg * hid),
            const(1, ndeg * hid),
            const(hid, hid),
            const(in2 - hid, hid),
            const(1, hid),
            const(hid, ncls),
            const(1, ncls),
        ],
        out_specs=pl.BlockSpec((mb, ncls), lambda i: (i, 0)),
        out_shape=jax.ShapeDtypeStruct((bm, ncls), jnp.float32),
    )(atoms, bonds_r, edges, w1a, w1b, b1c, w2a, w2b, b2c,
      woa, wob, bo2, Wf, bf2)


# speedup vs baseline: 71.5794x; 1.6611x over previous
import functools

import jax
import jax.numpy as jnp
from jax.experimental import pallas as pl


def _split(x):
    hi = x.astype(jnp.bfloat16)
    lo = (x - hi.astype(jnp.float32)).astype(jnp.bfloat16)
    return hi, lo


def _body(mb, na, nd, nf, nbf, hid, ncls,
          atoms_ref, bonds_ref, edges_ref,
          w1a_ref, b1_ref, w2a_ref, b2_ref,
          wball_ref, woa_ref, bo_ref,
          wf_ref, bf_ref, out_ref):
    ndeg = nd + 1
    atoms = atoms_ref[...]
    edges = edges_ref[...]
    bonds = bonds_ref[...]

    valid = edges != -1
    deg = jnp.sum(valid.astype(jnp.int32), axis=2)
    deg_col = deg.reshape(mb * na, 1)
    nz_col = (deg_col > 0).astype(jnp.float32)
    dlane = jax.lax.broadcasted_iota(jnp.int32, (mb * na, ndeg * hid), 1) // hid
    mask_cat = (dlane == deg_col).astype(jnp.float32)

    bsum = bonds[:, :, 0:nbf]
    for d in range(1, nd):
        bsum = bsum + bonds[:, :, d * nbf:(d + 1) * nbf]
    b6 = bsum.reshape(mb * na, nbf)
    bb_all = jnp.dot(b6, wball_ref[...],
                     preferred_element_type=jnp.float32)
    bb1 = bb_all[:, 0:ndeg * hid]
    bb2 = bb_all[:, ndeg * hid:2 * ndeg * hid]
    bbo = bb_all[:, 2 * ndeg * hid:]

    col = jax.lax.broadcasted_iota(jnp.int32, (mb, na, na), 2)
    onehots = []
    for d in range(nd):
        e_d = edges[:, :, d]
        onehots.append((e_d[:, :, None] == col).astype(jnp.bfloat16))
    adj = onehots[0]
    for d in range(1, nd):
        adj = adj + onehots[d]
    oh_cat = jnp.concatenate(onehots, axis=1)

    def bmm(a, b):
        return jax.lax.dot_general(
            a, b, (((2,), (1,)), ((0,), (0,))),
            preferred_element_type=jnp.float32)

    def gather2(oh, x):
        hi, lo = _split(x)
        return bmm(oh, hi) + bmm(oh, lo)

    def conv(x, wa_ref, b_ref, bb, in_dim):
        summed = x + gather2(adj, x)
        s = summed.reshape(mb * na, in_dim)
        y = (jnp.dot(s, wa_ref[...], preferred_element_type=jnp.float32)
             + bb + b_ref[...]) * mask_cat
        out = y[:, 0:hid]
        for d in range(1, ndeg):
            out = out + y[:, d * hid:(d + 1) * hid]
        return jnp.maximum(out, 0.0).reshape(mb, na, hid)

    def pool(x):
        g_all = gather2(oh_cat, x)
        p = x
        for d in range(nd):
            p = jnp.maximum(p, g_all[:, d * na:(d + 1) * na, :])
        return p * nz_col.reshape(mb, na, 1)

    x1 = conv(atoms, w1a_ref, b1_ref, bb1, nf)
    p1 = pool(x1)
    x2 = conv(p1, w2a_ref, b2_ref, bb2, hid)
    p2 = pool(x2)

    f = jnp.tanh(jnp.dot(p2.reshape(mb * na, hid), woa_ref[...],
                         preferred_element_type=jnp.float32)
                 + bbo + bo_ref[...])
    f = f * nz_col
    fp = f.reshape(mb, na, hid).sum(axis=1)
    logits = (jnp.dot(fp, wf_ref[...], preferred_element_type=jnp.float32)
              + bf_ref[...])
    out_ref[...] = jax.nn.sigmoid(logits)


@jax.jit
def kernel(atoms, bonds, edges, W1, b1, W2, b2, Wo, bo, Wf, bf):
    bm, na, nf = atoms.shape
    nd = edges.shape[-1]
    nbf = bonds.shape[-1]
    ndeg, in1, hid = W1.shape
    ncls = Wf.shape[-1]
    mb = 8
    grid = (bm // mb,)

    bonds_r = bonds.reshape(bm, na, nd * nbf)
    w1c = W1.transpose(1, 0, 2).reshape(in1, ndeg * hid)
    w1a, w1b = w1c[:nf], w1c[nf:]
    b1c = b1.reshape(1, ndeg * hid)
    in2 = W2.shape[1]
    w2c = W2.transpose(1, 0, 2).reshape(in2, ndeg * hid)
    w2a, w2b = w2c[:hid], w2c[hid:]
    b2c = b2.reshape(1, ndeg * hid)
    woa, wob = Wo[:hid], Wo[hid:]
    wball = jnp.concatenate([w1b, w2b, wob], axis=1)
    bo2 = bo.reshape(1, hid)
    bf2 = bf.reshape(1, ncls)

    const = lambda *shape: pl.BlockSpec(shape, lambda i: (0,) * len(shape))
    return pl.pallas_call(
        functools.partial(_body, mb, na, nd, nf, nbf, hid, ncls),
        grid=grid,
        in_specs=[
            pl.BlockSpec((mb, na, nf), lambda i: (i, 0, 0)),
            pl.BlockSpec((mb, na, nd * nbf), lambda i: (i, 0, 0)),
            pl.BlockSpec((mb, na, nd), lambda i: (i, 0, 0)),
            const(nf, ndeg * hid),
            const(1, ndeg * hid),
            const(hid, ndeg * hid),
            const(1, ndeg * hid),
            const(nbf, 2 * ndeg * hid + hid),
            const(hid, hid),
            const(1, hid),
            const(hid, ncls),
            const(1, ncls),
        ],
        out_specs=pl.BlockSpec((mb, ncls), lambda i: (i, 0)),
        out_shape=jax.ShapeDtypeStruct((bm, ncls), jnp.float32),
    )(atoms, bonds_r, edges, w1a, b1c, w2a, b2c,
      wball, woa, bo2, Wf, bf2)
